# baseline (speedup 1.0000x reference)
import jax
import jax.numpy as jnp
from jax.experimental import pallas as pl

T = 128
K = 8
E = 64
D = 1024
F = 1024


EPB = 2


def _moe_body(x_ref, ids_ref, w1_ref, s1_ref, w2_ref, s2_ref,
              smooth_ref, escale_ref, out_ref):
    i = pl.program_id(0)
    x = x_ref[...]
    contrib = None
    for j in range(EPB):
        e = i * EPB + j
        h = jnp.dot(x, w1_ref[j], preferred_element_type=jnp.float32)
        h = h * s1_ref[j]
        gate = h[:, :F]
        up = h[:, F:]
        act = (gate * jax.nn.sigmoid(gate)) * up
        act = act * smooth_ref[j]
        y = jnp.dot(act, w2_ref[j], preferred_element_type=jnp.float32)
        y = y * s2_ref[j]
        coef = jnp.sum(
            jnp.where(ids_ref[...] == e, escale_ref[...], 0.0), axis=1)
        c = coef[:, None] * y
        contrib = c if contrib is None else contrib + c

    @pl.when(i == 0)
    def _init():
        out_ref[...] = contrib

    @pl.when(i != 0)
    def _acc():
        out_ref[...] += contrib


def kernel(x, expert_ids, gmm1_weight, gmm1_weight_scale, gmm2_weight,
           gmm2_weight_scale, smooth_scales, expert_scales):
    return pl.pallas_call(
        _moe_body,
        grid=(E // EPB,),
        in_specs=[
            pl.BlockSpec((T, D), lambda e: (0, 0)),
            pl.BlockSpec((T, K), lambda e: (0, 0)),
            pl.BlockSpec((EPB, D, 2 * F), lambda e: (e, 0, 0)),
            pl.BlockSpec((EPB, 1, 2 * F), lambda e: (e, 0, 0)),
            pl.BlockSpec((EPB, F, D), lambda e: (e, 0, 0)),
            pl.BlockSpec((EPB, 1, D), lambda e: (e, 0, 0)),
            pl.BlockSpec((EPB, 1, F), lambda e: (e, 0, 0)),
            pl.BlockSpec((T, K), lambda e: (0, 0)),
        ],
        out_specs=pl.BlockSpec((T, D), lambda e: (0, 0)),
        out_shape=jax.ShapeDtypeStruct((T, D), jnp.float32),
    )(x, expert_ids, gmm1_weight, gmm1_weight_scale[:, None, :], gmm2_weight,
      gmm2_weight_scale[:, None, :], smooth_scales[:, None, :], expert_scales)


# speedup vs baseline: 1.0167x; 1.0167x over previous
import jax
import jax.numpy as jnp
from jax.experimental import pallas as pl

T = 128
K = 8
E = 64
D = 1024
F = 1024
C = F // 2


def _moe_body(x_ref, ids_ref, w1g0_ref, w1g1_ref, w1u0_ref, w1u1_ref,
              s1_ref, w2a_ref, w2b_ref, s2_ref, smooth_ref, escale_ref,
              out_ref):
    e = pl.program_id(0)
    x = x_ref[...]
    s1 = s1_ref[0]
    smooth = smooth_ref[0]

    g0 = jnp.dot(x, w1g0_ref[0], preferred_element_type=jnp.float32)
    g1 = jnp.dot(x, w1g1_ref[0], preferred_element_type=jnp.float32)
    u0 = jnp.dot(x, w1u0_ref[0], preferred_element_type=jnp.float32)
    u1 = jnp.dot(x, w1u1_ref[0], preferred_element_type=jnp.float32)

    gate0 = g0 * s1[:, 0:C]
    gate1 = g1 * s1[:, C:F]
    up0 = u0 * s1[:, F:F + C]
    up1 = u1 * s1[:, F + C:]
    act0 = (gate0 * jax.nn.sigmoid(gate0)) * up0 * smooth[:, 0:C]
    act1 = (gate1 * jax.nn.sigmoid(gate1)) * up1 * smooth[:, C:]

    y = jnp.dot(act0, w2a_ref[0], preferred_element_type=jnp.float32)
    y = y + jnp.dot(act1, w2b_ref[0], preferred_element_type=jnp.float32)
    y = y * s2_ref[0]

    coef = jnp.sum(
        jnp.where(ids_ref[...] == e, escale_ref[...], 0.0), axis=1)
    contrib = coef[:, None] * y

    @pl.when(e == 0)
    def _init():
        out_ref[...] = contrib

    @pl.when(e != 0)
    def _acc():
        out_ref[...] += contrib


def kernel(x, expert_ids, gmm1_weight, gmm1_weight_scale, gmm2_weight,
           gmm2_weight_scale, smooth_scales, expert_scales):
    return pl.pallas_call(
        _moe_body,
        grid=(E,),
        in_specs=[
            pl.BlockSpec((T, D), lambda e: (0, 0)),
            pl.BlockSpec((T, K), lambda e: (0, 0)),
            pl.BlockSpec((1, D, C), lambda e: (e, 0, 0)),
            pl.BlockSpec((1, D, C), lambda e: (e, 0, 1)),
            pl.BlockSpec((1, D, C), lambda e: (e, 0, 2)),
            pl.BlockSpec((1, D, C), lambda e: (e, 0, 3)),
            pl.BlockSpec((1, 1, 2 * F), lambda e: (e, 0, 0)),
            pl.BlockSpec((1, C, D), lambda e: (e, 0, 0)),
            pl.BlockSpec((1, C, D), lambda e: (e, 1, 0)),
            pl.BlockSpec((1, 1, D), lambda e: (e, 0, 0)),
            pl.BlockSpec((1, 1, F), lambda e: (e, 0, 0)),
            pl.BlockSpec((T, K), lambda e: (0, 0)),
        ],
        out_specs=pl.BlockSpec((T, D), lambda e: (0, 0)),
        out_shape=jax.ShapeDtypeStruct((T, D), jnp.float32),
    )(x, expert_ids, gmm1_weight, gmm1_weight, gmm1_weight, gmm1_weight,
      gmm1_weight_scale[:, None, :], gmm2_weight, gmm2_weight,
      gmm2_weight_scale[:, None, :], smooth_scales[:, None, :], expert_scales)
